# baseline (speedup 1.0000x reference)
import functools
import math

import jax
import jax.numpy as jnp
from jax.experimental import pallas as pl
from jax.experimental.pallas import tpu as pltpu

_NUM_UNITS = 1024
_SCALE = float(_NUM_UNITS) ** 0.5
_NEG2LOG1E4_OVER_D = -2.0 * math.log(10000.0) / _NUM_UNITS
_HALF_PI = math.pi / 2.0
_SUB = 64


def _pe_body(out_ref, s64_ref, c64_ref, *, block_t: int, batch: int):
    pid = pl.program_id(0)

    @pl.when(pid == 0)
    def _init_tables():
        r = jax.lax.broadcasted_iota(jnp.int32, (_SUB, _NUM_UNITS), 0)
        ch = jax.lax.broadcasted_iota(jnp.int32, (_SUB, _NUM_UNITS), 1)
        rw = r.astype(jnp.float32) * jnp.exp(
            ch.astype(jnp.float32) * _NEG2LOG1E4_OVER_D
        )
        s64_ref[...] = jnp.sin(rw)
        c64_ref[...] = jnp.sin(rw + _HALF_PI)

    ch1 = jax.lax.broadcasted_iota(jnp.int32, (1, _NUM_UNITS), 1)
    w1 = jnp.exp(ch1.astype(jnp.float32) * _NEG2LOG1E4_OVER_D)
    off = (ch1 % 2).astype(jnp.float32) * _HALF_PI
    s64 = s64_ref[...]
    c64 = c64_ref[...]
    for a in range(block_t // _SUB):
        p0f = (pid * block_t + a * _SUB).astype(jnp.float32)
        phase = p0f * w1 + off
        sb = jnp.sin(phase) * _SCALE
        cb = jnp.sin(phase + _HALF_PI) * _SCALE
        pe = sb * c64 + cb * s64
        out_ref[:, a * _SUB : (a + 1) * _SUB, :] = jnp.broadcast_to(
            pe[None], (batch, _SUB, _NUM_UNITS)
        )

    @pl.when(pid == 0)
    def _zero_first_row():
        out_ref[:, 0:1, :] = jnp.zeros((batch, 1, _NUM_UNITS), jnp.float32)


def kernel(inputs):
    n, t, d = inputs.shape
    block_t = 256
    body = functools.partial(_pe_body, block_t=block_t, batch=n)
    return pl.pallas_call(
        body,
        grid=(t // block_t,),
        out_shape=jax.ShapeDtypeStruct((n, t, d), jnp.float32),
        out_specs=pl.BlockSpec((n, block_t, d), lambda i: (0, i, 0)),
        scratch_shapes=[
            pltpu.VMEM((_SUB, d), jnp.float32),
            pltpu.VMEM((_SUB, d), jnp.float32),
        ],
    )()


# speedup vs baseline: 1.1464x; 1.1464x over previous
import functools
import math

import jax
import jax.numpy as jnp
from jax.experimental import pallas as pl
from jax.experimental.pallas import tpu as pltpu

_NUM_UNITS = 1024
_SCALE = float(_NUM_UNITS) ** 0.5
_NEG2LOG1E4_OVER_D = -2.0 * math.log(10000.0) / _NUM_UNITS
_HALF_PI = math.pi / 2.0
_SUB = 64
_STRIPE = 512


def _pe_body(out_ref, table_ref, s64_ref, c64_ref, sem_ref, *, seq: int, batch: int):
    r = jax.lax.broadcasted_iota(jnp.int32, (_SUB, _NUM_UNITS), 0)
    ch = jax.lax.broadcasted_iota(jnp.int32, (_SUB, _NUM_UNITS), 1)
    rw = r.astype(jnp.float32) * jnp.exp(ch.astype(jnp.float32) * _NEG2LOG1E4_OVER_D)
    s64_ref[...] = jnp.sin(rw)
    c64_ref[...] = jnp.sin(rw + _HALF_PI)

    ch1 = jax.lax.broadcasted_iota(jnp.int32, (1, _NUM_UNITS), 1)
    w1 = jnp.exp(ch1.astype(jnp.float32) * _NEG2LOG1E4_OVER_D)
    off = (ch1 % 2).astype(jnp.float32) * _HALF_PI
    s64 = s64_ref[...]
    c64 = c64_ref[...]

    n_stripes = seq // _STRIPE

    def _copies(s):
        return [
            pltpu.make_async_copy(
                table_ref.at[pl.ds(s * _STRIPE, _STRIPE), :],
                out_ref.at[n, pl.ds(s * _STRIPE, _STRIPE), :],
                sem_ref.at[s % 2, n],
            )
            for n in range(batch)
        ]

    for s in range(n_stripes):
        for a in range(_STRIPE // _SUB):
            p0 = s * _STRIPE + a * _SUB
            phase = float(p0) * w1 + off
            sb = jnp.sin(phase) * _SCALE
            cb = jnp.sin(phase + _HALF_PI) * _SCALE
            row = pl.ds(p0, _SUB)
            table_ref[row, :] = sb * c64 + cb * s64
        if s == 0:
            table_ref[0:1, :] = jnp.zeros((1, _NUM_UNITS), jnp.float32)
        if s >= 2:
            for c in _copies(s - 2):
                c.wait()
        for c in _copies(s):
            c.start()
    for s in (n_stripes - 2, n_stripes - 1):
        for c in _copies(s):
            c.wait()


def kernel(inputs):
    n, t, d = inputs.shape
    body = functools.partial(_pe_body, seq=t, batch=n)
    return pl.pallas_call(
        body,
        out_shape=jax.ShapeDtypeStruct((n, t, d), jnp.float32),
        out_specs=pl.BlockSpec(memory_space=pl.ANY),
        scratch_shapes=[
            pltpu.VMEM((t, d), jnp.float32),
            pltpu.VMEM((_SUB, d), jnp.float32),
            pltpu.VMEM((_SUB, d), jnp.float32),
            pltpu.SemaphoreType.DMA((2, n)),
        ],
    )()
